# baseline (speedup 1.0000x reference)
import functools

import jax
import jax.numpy as jnp
from jax import lax
from jax.experimental import pallas as pl
from jax.experimental.pallas import tpu as pltpu
from jax.experimental.pallas import tpu_sc as plsc


def _make_sc_add(S, D, B, R, KREG):
    info = plsc.get_sparse_core_info()
    NC, NS = info.num_cores, info.num_subcores
    NW = NC * NS
    rspan = S // NW
    T = rspan // R
    NK = R * D // (16 * KREG)
    assert S % NW == 0 and rspan % R == 0 and T % 2 == 0
    assert R * D % (16 * KREG) == 0 and D % (16 * KREG) == 0
    mesh = plsc.VectorSubcoreMesh(core_axis_name="c", subcore_axis_name="s")

    @functools.partial(
        pl.kernel,
        mesh=mesh,
        out_type=jax.ShapeDtypeStruct((B * S, D), jnp.float32),
        scratch_types=(
            [pltpu.VMEM((R, D), jnp.float32) for _ in range(2 * B)]
            + [pltpu.VMEM((R, D), jnp.float32) for _ in range(B)]
            + [pltpu.VMEM((R, D), jnp.float32) for _ in range(2)]
            + [
                pltpu.SemaphoreType.DMA,
                pltpu.SemaphoreType.DMA,
                pltpu.SemaphoreType.DMA,
                pltpu.SemaphoreType.DMA,
                pltpu.SemaphoreType.DMA,
            ]
        ),
    )
    def k(x_hbm, e_hbm, o_hbm, *refs):
        xin = tuple(tuple(refs[b * 2 + p] for p in range(2)) for b in range(B))
        xout = tuple(refs[2 * B + b] for b in range(B))
        eb = (refs[3 * B], refs[3 * B + 1])
        sxl = (refs[3 * B + 2], refs[3 * B + 3])
        se = (refs[3 * B + 4], refs[3 * B + 5])
        sso = refs[3 * B + 6]
        wid = lax.axis_index("s") * NC + lax.axis_index("c")
        wrbase = wid * rspan

        def xrow(t, b):
            return b * S + wrbase + t * R

        def erow(t):
            return wrbase + t * R

        def xload(t, b, p):
            pltpu.make_async_copy(
                x_hbm.at[pl.ds(xrow(t, b), R)], xin[b][p], sxl[p]).start()

        def eload(t, p):
            pltpu.make_async_copy(
                e_hbm.at[pl.ds(erow(t), R)], eb[p], se[p]).start()

        for p in range(2):
            eload(p, p)
            for b in range(B):
                xload(p, b, p)

        def body(it, carry):
            t0 = it * 2
            for tt in range(2):
                t = t0 + tt
                pltpu.make_async_copy(
                    e_hbm.at[pl.ds(erow(t), R)], eb[tt], se[tt]).wait()
                for b in range(B):
                    pltpu.make_async_copy(
                        x_hbm.at[pl.ds(xrow(t, b), R)], xin[b][tt],
                        sxl[tt]).wait()

                def drain():
                    for b in range(B):
                        pltpu.make_async_copy(
                            xout[b], o_hbm.at[pl.ds(0, R)], sso).wait()

                if tt == 0:
                    pl.when(t0 > 0)(drain)
                else:
                    drain()

                @plsc.parallel_loop(0, NK, unroll=2)
                def addk(kk):
                    r = kk // (D // (16 * KREG))
                    c0 = (kk % (D // (16 * KREG))) * (16 * KREG)
                    evals = [
                        eb[tt][r, pl.ds(c0 + j * 16, 16)] for j in range(KREG)
                    ]
                    for b in range(B):
                        for j in range(KREG):
                            sl = pl.ds(c0 + j * 16, 16)
                            xout[b][r, sl] = xin[b][tt][r, sl] + evals[j]

                for b in range(B):
                    pltpu.make_async_copy(
                        xout[b], o_hbm.at[pl.ds(xrow(t, b), R)], sso).start()

                def prefetch():
                    eload(t + 2, tt)
                    for b in range(B):
                        xload(t + 2, b, tt)

                pl.when(t + 2 < T)(prefetch)
            return carry

        lax.fori_loop(0, T // 2, body, 0)

        for b in range(B):
            pltpu.make_async_copy(xout[b], o_hbm.at[pl.ds(0, R)], sso).wait()

    return k


def kernel(x, emb):
    B, S, D = x.shape
    k = _make_sc_add(S, D, B, 8, 8)
    out = k(x.reshape(B * S, D), emb[:S])
    return out.reshape(B, S, D)


# speedup vs baseline: 1.0231x; 1.0231x over previous
import functools

import jax
import jax.numpy as jnp
from jax import lax
from jax.experimental import pallas as pl
from jax.experimental.pallas import tpu as pltpu
from jax.experimental.pallas import tpu_sc as plsc


def _make_sc_add(S, D, B, R, KREG):
    info = plsc.get_sparse_core_info()
    NC, NS = info.num_cores, info.num_subcores
    NW = NC * NS
    rspan = S // NW
    T = rspan // R
    NK = R * D // (16 * KREG)
    assert S % NW == 0 and rspan % R == 0 and T % 2 == 0
    assert R * D % (16 * KREG) == 0 and D % (16 * KREG) == 0
    mesh = plsc.VectorSubcoreMesh(core_axis_name="c", subcore_axis_name="s")

    @functools.partial(
        pl.kernel,
        mesh=mesh,
        out_type=jax.ShapeDtypeStruct((B * S, D), jnp.float32),
        scratch_types=(
            [pltpu.VMEM((R, D), jnp.float32) for _ in range(2 * B)]
            + [pltpu.VMEM((R, D), jnp.float32) for _ in range(B)]
            + [pltpu.VMEM((R, D), jnp.float32) for _ in range(2)]
            + [
                pltpu.SemaphoreType.DMA,
                pltpu.SemaphoreType.DMA,
                pltpu.SemaphoreType.DMA,
                pltpu.SemaphoreType.DMA,
                pltpu.SemaphoreType.DMA,
            ]
        ),
    )
    def k(x_hbm, e_hbm, o_hbm, *refs):
        xin = tuple(tuple(refs[b * 2 + p] for p in range(2)) for b in range(B))
        xout = tuple(refs[2 * B + b] for b in range(B))
        eb = (refs[3 * B], refs[3 * B + 1])
        sxl = (refs[3 * B + 2], refs[3 * B + 3])
        se = (refs[3 * B + 4], refs[3 * B + 5])
        sso = refs[3 * B + 6]
        wid = lax.axis_index("s") * NC + lax.axis_index("c")
        wrbase = wid * rspan

        def xrow(t, b):
            return b * S + wrbase + t * R

        def erow(t):
            return wrbase + t * R

        def xload(t, b, p):
            pltpu.make_async_copy(
                x_hbm.at[pl.ds(xrow(t, b), R)], xin[b][p], sxl[p]).start()

        def eload(t, p):
            pltpu.make_async_copy(
                e_hbm.at[pl.ds(erow(t), R)], eb[p], se[p]).start()

        for p in range(2):
            eload(p, p)
            for b in range(B):
                xload(p, b, p)

        def body(it, carry):
            t0 = it * 2
            for tt in range(2):
                t = t0 + tt
                pltpu.make_async_copy(
                    e_hbm.at[pl.ds(erow(t), R)], eb[tt], se[tt]).wait()
                for b in range(B):
                    pltpu.make_async_copy(
                        x_hbm.at[pl.ds(xrow(t, b), R)], xin[b][tt],
                        sxl[tt]).wait()

                def drain():
                    for b in range(B):
                        pltpu.make_async_copy(
                            xout[b], o_hbm.at[pl.ds(0, R)], sso).wait()

                if tt == 0:
                    pl.when(t0 > 0)(drain)
                else:
                    drain()

                @plsc.parallel_loop(0, NK, unroll=1)
                def addk(kk):
                    r = kk // (D // (16 * KREG))
                    c0 = (kk % (D // (16 * KREG))) * (16 * KREG)
                    evals = [
                        eb[tt][r, pl.ds(c0 + j * 16, 16)] for j in range(KREG)
                    ]
                    for b in range(B):
                        for j in range(KREG):
                            sl = pl.ds(c0 + j * 16, 16)
                            xout[b][r, sl] = xin[b][tt][r, sl] + evals[j]

                for b in range(B):
                    pltpu.make_async_copy(
                        xout[b], o_hbm.at[pl.ds(xrow(t, b), R)], sso).start()

                def prefetch():
                    eload(t + 2, tt)
                    for b in range(B):
                        xload(t + 2, b, tt)

                pl.when(t + 2 < T)(prefetch)
            return carry

        lax.fori_loop(0, T // 2, body, 0)

        for b in range(B):
            pltpu.make_async_copy(xout[b], o_hbm.at[pl.ds(0, R)], sso).wait()

    return k


def kernel(x, emb):
    B, S, D = x.shape
    k = _make_sc_add(S, D, B, 8, 8)
    out = k(x.reshape(B * S, D), emb[:S])
    return out.reshape(B, S, D)
